# baseline (speedup 1.0000x reference)
import functools

import jax
import jax.numpy as jnp
from jax import lax
from jax.experimental import pallas as pl
from jax.experimental.pallas import tpu as pltpu
from jax.experimental.pallas import tpu_sc as plsc

B = 1024
DB = 16384
DML_DIM = 512
EMBED_DIM = 2048
OT_DIM = 384
OUT_DIM = 24960
IN_FLAT = 3072
NUM_S = 10

_BT_A = 128


def _prep_body(xf_ref, wd_ref, we_ref, cls_ref, emb_ref):
    xf = xf_ref[...]
    cls_ref[...] = jnp.dot(xf, wd_ref[...], preferred_element_type=jnp.float32)
    mu = jnp.mean(xf, axis=1, keepdims=True)
    var = jnp.mean((xf - mu) ** 2, axis=1, keepdims=True)
    xn = (xf - mu) / (jnp.sqrt(var) + 1e-6)
    emb_ref[...] = jnp.dot(xn, we_ref[...], preferred_element_type=jnp.float32)


def _prep(xf, W_dml, W_emb):
    return pl.pallas_call(
        _prep_body,
        grid=(B // _BT_A,),
        in_specs=[
            pl.BlockSpec((_BT_A, IN_FLAT), lambda i: (i, 0)),
            pl.BlockSpec((IN_FLAT, DML_DIM), lambda i: (0, 0)),
            pl.BlockSpec((IN_FLAT, EMBED_DIM), lambda i: (0, 0)),
        ],
        out_specs=[
            pl.BlockSpec((_BT_A, DML_DIM), lambda i: (i, 0)),
            pl.BlockSpec((_BT_A, EMBED_DIM), lambda i: (i, 0)),
        ],
        out_shape=[
            jax.ShapeDtypeStruct((B, DML_DIM), jnp.float32),
            jax.ShapeDtypeStruct((B, EMBED_DIM), jnp.float32),
        ],
    )(xf, W_dml, W_emb)


_BT_B = 128


_CB = 2048
_NCB = DB // _CB


def _topk_body(cls_ref, dml_ref, idx_ref, s_ref):
    c = pl.program_id(1)
    cls = cls_ref[...]
    INF = jnp.float32(jnp.inf)
    BIG = jnp.int32(DB)

    dml_c = dml_ref[...]
    k2 = jnp.sum(dml_c * dml_c, axis=1, keepdims=True)
    dots = lax.dot_general(dml_c, cls, (((1,), (1,)), ((), ())),
                           preferred_element_type=jnp.float32)
    s_ref[c] = k2 - 2.0 * dots

    @pl.when(c == _NCB - 1)
    def _extract_all():
        siota = lax.broadcasted_iota(jnp.int32, (_CB, _BT_B), 0)

        def extract(r, carry):
            pv, pi, acc = carry

            def scan_chunk(cc, carry2):
                bm, bi = carry2
                s = s_ref[cc]
                gi = siota + cc * _CB
                ok = (s > pv) | ((s == pv) & (gi > pi))
                masked = jnp.where(ok, s, INF)
                m = jnp.min(masked, axis=0, keepdims=True)
                i_c = jnp.min(jnp.where(masked == m, gi, BIG), axis=0,
                              keepdims=True)
                take = (m < bm) | ((m == bm) & (i_c < bi))
                return jnp.where(take, m, bm), jnp.where(take, i_c, bi)

            bm, bi = lax.fori_loop(0, _NCB, scan_chunk,
                                   (jnp.full((1, _BT_B), INF),
                                    jnp.full((1, _BT_B), BIG)))
            acc_iota = lax.broadcasted_iota(jnp.int32, (16, _BT_B), 0)
            acc = jnp.where(acc_iota == r, bi, acc)
            return bm, bi, acc

        _, _, acc = lax.fori_loop(
            0, NUM_S, extract,
            (jnp.full((1, _BT_B), -INF), jnp.full((1, _BT_B), -1, jnp.int32),
             jnp.zeros((16, _BT_B), jnp.int32)))
        idx_ref[...] = jnp.transpose(acc)[:, :NUM_S]


def _topk(cls, dml):
    return pl.pallas_call(
        _topk_body,
        grid=(B // _BT_B, _NCB),
        in_specs=[
            pl.BlockSpec((_BT_B, DML_DIM), lambda i, j: (i, 0)),
            pl.BlockSpec((_CB, DML_DIM), lambda i, j: (j, 0)),
        ],
        out_specs=pl.BlockSpec((_BT_B, NUM_S), lambda i, j: (i, 0)),
        out_shape=jax.ShapeDtypeStruct((B, NUM_S), jnp.int32),
        scratch_shapes=[pltpu.VMEM((_NCB, _CB, _BT_B), jnp.float32)],
    )(cls, dml)


_NW = 32
_BPW = B // _NW
_CH = 8
_NCHUNK = _BPW // _CH
_ROWS = _CH * NUM_S
_LC = OT_DIM // 16


def _gather_sum(OTinput, OToutput, idx3d):
    mesh = plsc.VectorSubcoreMesh(core_axis_name="c", subcore_axis_name="s")

    @functools.partial(
        pl.kernel,
        mesh=mesh,
        out_type=jax.ShapeDtypeStruct((B, OT_DIM), jnp.float32),
        scratch_types=[
            pltpu.VMEM((_NCHUNK, _ROWS), jnp.int32),
            pltpu.VMEM((_ROWS, OT_DIM), jnp.float32),
            pltpu.VMEM((_BPW, OT_DIM), jnp.float32),
            pltpu.SemaphoreType.DMA,
        ],
    )
    def k(otin_hbm, otout_hbm, idx_hbm, out_hbm, idx_v, rows_v, acc_v, sem):
        wid = lax.axis_index("c") * 16 + lax.axis_index("s")
        pltpu.sync_copy(idx_hbm.at[wid], idx_v)

        def zero_row(i, carry):
            for j in range(_LC):
                acc_v[i, pl.ds(j * 16, 16)] = jnp.zeros((16,), jnp.float32)
            return carry

        lax.fori_loop(0, _BPW, zero_row, 0)

        for tbl in (otin_hbm, otout_hbm):
            def chunk_body(c, carry, tbl=tbl):
                pltpu.async_copy(tbl.at[idx_v.at[c]], rows_v, sem).wait()

                def q_body(b, carry2):
                    r0 = b * NUM_S
                    for j in range(_LC):
                        sl = pl.ds(j * 16, 16)
                        v = rows_v[r0, sl]
                        for r in range(1, NUM_S):
                            v = v + rows_v[r0 + r, sl]
                        a = c * _CH + b
                        acc_v[a, sl] = acc_v[a, sl] + v
                    return carry2

                lax.fori_loop(0, _CH, q_body, 0)
                return carry

            lax.fori_loop(0, _NCHUNK, chunk_body, 0)

        pltpu.sync_copy(acc_v, out_hbm.at[pl.ds(wid * _BPW, _BPW)])

    return k(OTinput, OToutput, idx3d)


_NT_D = 640


def _proj1_body(emb_ref, w1_ref, out_ref):
    out_ref[...] = jnp.dot(emb_ref[...], w1_ref[...],
                           preferred_element_type=jnp.float32)


def _proj1(embed, W_cip1):
    return pl.pallas_call(
        _proj1_body,
        grid=(OUT_DIM // _NT_D,),
        in_specs=[
            pl.BlockSpec((B, EMBED_DIM), lambda j: (0, 0)),
            pl.BlockSpec((EMBED_DIM, _NT_D), lambda j: (0, j)),
        ],
        out_specs=pl.BlockSpec((B, _NT_D), lambda j: (0, j)),
        out_shape=jax.ShapeDtypeStruct((B, OUT_DIM), jnp.float32),
    )(embed, W_cip1)


def _proj2_body(part_ref, nbr_ref, w2_ref, out_ref):
    out_ref[...] = part_ref[...] + jnp.dot(
        nbr_ref[...], w2_ref[...], preferred_element_type=jnp.float32)


def _proj2(part, nbr, W_cip2):
    return pl.pallas_call(
        _proj2_body,
        grid=(OUT_DIM // _NT_D,),
        in_specs=[
            pl.BlockSpec((B, _NT_D), lambda j: (0, j)),
            pl.BlockSpec((B, OT_DIM), lambda j: (0, 0)),
            pl.BlockSpec((OT_DIM, _NT_D), lambda j: (0, j)),
        ],
        out_specs=pl.BlockSpec((B, _NT_D), lambda j: (0, j)),
        out_shape=jax.ShapeDtypeStruct((B, OUT_DIM), jnp.float32),
        input_output_aliases={0: 0},
    )(part, nbr, W_cip2)


def kernel(x, dmlfeature, OTinput, OToutput, W_dml, W_emb, W_cip1, W_cip2):
    xf = x.reshape(B, IN_FLAT)
    cls, embed = _prep(xf, W_dml, W_emb)
    idx = _topk(cls, dmlfeature)
    idx3d = idx.reshape(_NW, _NCHUNK, _ROWS)
    nbr = _gather_sum(OTinput, OToutput, idx3d)
    part = _proj1(embed, W_cip1)
    out = _proj2(part, nbr, W_cip2)
    return out.reshape(B, 65, OUT_DIM // 65)


# speedup vs baseline: 1.0226x; 1.0226x over previous
import functools

import jax
import jax.numpy as jnp
from jax import lax
from jax.experimental import pallas as pl
from jax.experimental.pallas import tpu as pltpu
from jax.experimental.pallas import tpu_sc as plsc

B = 1024
DB = 16384
DML_DIM = 512
EMBED_DIM = 2048
OT_DIM = 384
OUT_DIM = 24960
IN_FLAT = 3072
NUM_S = 10

_BT_A = 128


def _prep_body(xf_ref, wd_ref, we_ref, cls_ref, emb_ref):
    xf = xf_ref[...]
    cls_ref[...] = jnp.dot(xf, wd_ref[...], preferred_element_type=jnp.float32)
    mu = jnp.mean(xf, axis=1, keepdims=True)
    var = jnp.mean((xf - mu) ** 2, axis=1, keepdims=True)
    xn = (xf - mu) / (jnp.sqrt(var) + 1e-6)
    emb_ref[...] = jnp.dot(xn, we_ref[...], preferred_element_type=jnp.float32)


def _prep(xf, W_dml, W_emb):
    return pl.pallas_call(
        _prep_body,
        grid=(B // _BT_A,),
        in_specs=[
            pl.BlockSpec((_BT_A, IN_FLAT), lambda i: (i, 0)),
            pl.BlockSpec((IN_FLAT, DML_DIM), lambda i: (0, 0)),
            pl.BlockSpec((IN_FLAT, EMBED_DIM), lambda i: (0, 0)),
        ],
        out_specs=[
            pl.BlockSpec((_BT_A, DML_DIM), lambda i: (i, 0)),
            pl.BlockSpec((_BT_A, EMBED_DIM), lambda i: (i, 0)),
        ],
        out_shape=[
            jax.ShapeDtypeStruct((B, DML_DIM), jnp.float32),
            jax.ShapeDtypeStruct((B, EMBED_DIM), jnp.float32),
        ],
    )(xf, W_dml, W_emb)


_BT_B = 128


_CB = 2048
_NCB = DB // _CB


def _topk_body(cls_ref, dml_ref, idx_ref, s_ref):
    c = pl.program_id(1)
    cls = cls_ref[...]
    INF = jnp.float32(jnp.inf)
    BIG = jnp.int32(DB)

    dml_c = dml_ref[...]
    k2 = jnp.sum(dml_c * dml_c, axis=1, keepdims=True)
    dots = lax.dot_general(dml_c, cls, (((1,), (1,)), ((), ())),
                           preferred_element_type=jnp.float32)
    s_ref[c] = k2 - 2.0 * dots

    @pl.when(c == _NCB - 1)
    def _extract_all():
        siota = lax.broadcasted_iota(jnp.int32, (_CB, _BT_B), 0)

        def extract(r, carry):
            pv, pi, acc = carry

            def scan_chunk(cc, carry2):
                bm, bi = carry2
                s = s_ref[cc]
                gi = siota + cc * _CB
                ok = (s > pv) | ((s == pv) & (gi > pi))
                masked = jnp.where(ok, s, INF)
                m = jnp.min(masked, axis=0, keepdims=True)
                i_c = jnp.min(jnp.where(masked == m, gi, BIG), axis=0,
                              keepdims=True)
                take = (m < bm) | ((m == bm) & (i_c < bi))
                return jnp.where(take, m, bm), jnp.where(take, i_c, bi)

            bm, bi = lax.fori_loop(0, _NCB, scan_chunk,
                                   (jnp.full((1, _BT_B), INF),
                                    jnp.full((1, _BT_B), BIG)))
            acc_iota = lax.broadcasted_iota(jnp.int32, (16, _BT_B), 0)
            acc = jnp.where(acc_iota == r, bi, acc)
            return bm, bi, acc

        _, _, acc = lax.fori_loop(
            0, NUM_S, extract,
            (jnp.full((1, _BT_B), -INF), jnp.full((1, _BT_B), -1, jnp.int32),
             jnp.zeros((16, _BT_B), jnp.int32)))
        idx_ref[...] = jnp.transpose(acc)[:, :NUM_S]


def _topk(cls, dml):
    return pl.pallas_call(
        _topk_body,
        grid=(B // _BT_B, _NCB),
        in_specs=[
            pl.BlockSpec((_BT_B, DML_DIM), lambda i, j: (i, 0)),
            pl.BlockSpec((_CB, DML_DIM), lambda i, j: (j, 0)),
        ],
        out_specs=pl.BlockSpec((_BT_B, NUM_S), lambda i, j: (i, 0)),
        out_shape=jax.ShapeDtypeStruct((B, NUM_S), jnp.int32),
        scratch_shapes=[pltpu.VMEM((_NCB, _CB, _BT_B), jnp.float32)],
    )(cls, dml)


_NW = 32
_BPW = B // _NW
_CH = 8
_NCHUNK = _BPW // _CH
_ROWS = _CH * NUM_S
_LC = OT_DIM // 16


def _gather_sum(OTinput, OToutput, idx3d):
    mesh = plsc.VectorSubcoreMesh(core_axis_name="c", subcore_axis_name="s")

    @functools.partial(
        pl.kernel,
        mesh=mesh,
        out_type=jax.ShapeDtypeStruct((B, OT_DIM), jnp.float32),
        scratch_types=[
            pltpu.VMEM((_NCHUNK, _ROWS), jnp.int32),
            pltpu.VMEM((_ROWS, OT_DIM), jnp.float32),
            pltpu.VMEM((_BPW, OT_DIM), jnp.float32),
            pltpu.SemaphoreType.DMA,
        ],
    )
    def k(otin_hbm, otout_hbm, idx_hbm, out_hbm, idx_v, rows_v, acc_v, sem):
        wid = lax.axis_index("c") * 16 + lax.axis_index("s")
        pltpu.sync_copy(idx_hbm.at[wid], idx_v)

        def zero_row(i, carry):
            for j in range(_LC):
                acc_v[i, pl.ds(j * 16, 16)] = jnp.zeros((16,), jnp.float32)
            return carry

        lax.fori_loop(0, _BPW, zero_row, 0)

        for tbl in (otin_hbm, otout_hbm):
            def chunk_body(c, carry, tbl=tbl):
                pltpu.async_copy(tbl.at[idx_v.at[c]], rows_v, sem).wait()

                def q_body(b, carry2):
                    r0 = b * NUM_S
                    for j in range(_LC):
                        sl = pl.ds(j * 16, 16)
                        v = rows_v[r0, sl]
                        for r in range(1, NUM_S):
                            v = v + rows_v[r0 + r, sl]
                        a = c * _CH + b
                        acc_v[a, sl] = acc_v[a, sl] + v
                    return carry2

                lax.fori_loop(0, _CH, q_body, 0)
                return carry

            lax.fori_loop(0, _NCHUNK, chunk_body, 0)

        pltpu.sync_copy(acc_v, out_hbm.at[pl.ds(wid * _BPW, _BPW)])

    return k(OTinput, OToutput, idx3d)


_NT_D = 640


def _proj_body(emb_ref, nbr_ref, w1_ref, w2_ref, out_ref):
    out_ref[...] = (
        jnp.dot(emb_ref[...], w1_ref[...], preferred_element_type=jnp.float32)
        + jnp.dot(nbr_ref[...], w2_ref[...], preferred_element_type=jnp.float32)
    )


def _proj(embed, nbr, W_cip1, W_cip2):
    return pl.pallas_call(
        _proj_body,
        grid=(OUT_DIM // _NT_D,),
        in_specs=[
            pl.BlockSpec((B, EMBED_DIM), lambda j: (0, 0)),
            pl.BlockSpec((B, OT_DIM), lambda j: (0, 0)),
            pl.BlockSpec((EMBED_DIM, _NT_D), lambda j: (0, j)),
            pl.BlockSpec((OT_DIM, _NT_D), lambda j: (0, j)),
        ],
        out_specs=pl.BlockSpec((B, _NT_D), lambda j: (0, j)),
        out_shape=jax.ShapeDtypeStruct((B, OUT_DIM), jnp.float32),
    )(embed, nbr, W_cip1, W_cip2)


def kernel(x, dmlfeature, OTinput, OToutput, W_dml, W_emb, W_cip1, W_cip2):
    xf = x.reshape(B, IN_FLAT)
    cls, embed = _prep(xf, W_dml, W_emb)
    idx = _topk(cls, dmlfeature)
    idx3d = idx.reshape(_NW, _NCHUNK, _ROWS)
    nbr = _gather_sum(OTinput, OToutput, idx3d)
    out = _proj(embed, nbr, W_cip1, W_cip2)
    return out.reshape(B, 65, OUT_DIM // 65)


# speedup vs baseline: 1.0597x; 1.0363x over previous
import functools

import jax
import jax.numpy as jnp
from jax import lax
from jax.experimental import pallas as pl
from jax.experimental.pallas import tpu as pltpu
from jax.experimental.pallas import tpu_sc as plsc

B = 1024
DB = 16384
DML_DIM = 512
EMBED_DIM = 2048
OT_DIM = 384
OUT_DIM = 24960
IN_FLAT = 3072
NUM_S = 10

_BT_A = 128


def _prep_body(xf_ref, wd_ref, we_ref, cls_ref, emb_ref):
    xf = xf_ref[...]
    cls_ref[...] = jnp.dot(xf, wd_ref[...], preferred_element_type=jnp.float32)
    mu = jnp.mean(xf, axis=1, keepdims=True)
    var = jnp.mean((xf - mu) ** 2, axis=1, keepdims=True)
    xn = (xf - mu) / (jnp.sqrt(var) + 1e-6)
    emb_ref[...] = jnp.dot(xn, we_ref[...], preferred_element_type=jnp.float32)


def _prep(xf, W_dml, W_emb):
    return pl.pallas_call(
        _prep_body,
        grid=(B // _BT_A,),
        in_specs=[
            pl.BlockSpec((_BT_A, IN_FLAT), lambda i: (i, 0)),
            pl.BlockSpec((IN_FLAT, DML_DIM), lambda i: (0, 0)),
            pl.BlockSpec((IN_FLAT, EMBED_DIM), lambda i: (0, 0)),
        ],
        out_specs=[
            pl.BlockSpec((_BT_A, DML_DIM), lambda i: (i, 0)),
            pl.BlockSpec((_BT_A, EMBED_DIM), lambda i: (i, 0)),
        ],
        out_shape=[
            jax.ShapeDtypeStruct((B, DML_DIM), jnp.float32),
            jax.ShapeDtypeStruct((B, EMBED_DIM), jnp.float32),
        ],
    )(xf, W_dml, W_emb)


_BT_B = 128


_CB = 2048
_NCB = DB // _CB


def _topk_body(cls_ref, dml_ref, idx_ref, s_ref):
    c = pl.program_id(1)
    cls = cls_ref[...]
    INF = jnp.float32(jnp.inf)
    BIG = jnp.int32(DB)

    dml_c = dml_ref[...]
    k2 = jnp.sum(dml_c * dml_c, axis=1, keepdims=True)
    dots = lax.dot_general(dml_c, cls, (((1,), (1,)), ((), ())),
                           preferred_element_type=jnp.float32)
    s_ref[c] = k2 - 2.0 * dots

    @pl.when(c == _NCB - 1)
    def _extract_all():
        siota = lax.broadcasted_iota(jnp.int32, (_CB, _BT_B), 0)

        def extract(r, carry):
            pv, pi, acc = carry

            def scan_chunk(cc, carry2):
                bm, bi = carry2
                s = s_ref[cc]
                gi = siota + cc * _CB
                ok = (s > pv) | ((s == pv) & (gi > pi))
                masked = jnp.where(ok, s, INF)
                m = jnp.min(masked, axis=0, keepdims=True)
                i_c = jnp.min(jnp.where(masked == m, gi, BIG), axis=0,
                              keepdims=True)
                take = (m < bm) | ((m == bm) & (i_c < bi))
                return jnp.where(take, m, bm), jnp.where(take, i_c, bi)

            bm, bi = lax.fori_loop(0, _NCB, scan_chunk,
                                   (jnp.full((1, _BT_B), INF),
                                    jnp.full((1, _BT_B), BIG)))
            acc_iota = lax.broadcasted_iota(jnp.int32, (16, _BT_B), 0)
            acc = jnp.where(acc_iota == r, bi, acc)
            return bm, bi, acc

        _, _, acc = lax.fori_loop(
            0, NUM_S, extract,
            (jnp.full((1, _BT_B), -INF), jnp.full((1, _BT_B), -1, jnp.int32),
             jnp.zeros((16, _BT_B), jnp.int32)))
        idx_ref[...] = jnp.transpose(acc)[:, :NUM_S]


def _topk(cls, dml):
    return pl.pallas_call(
        _topk_body,
        grid=(B // _BT_B, _NCB),
        in_specs=[
            pl.BlockSpec((_BT_B, DML_DIM), lambda i, j: (i, 0)),
            pl.BlockSpec((_CB, DML_DIM), lambda i, j: (j, 0)),
        ],
        out_specs=pl.BlockSpec((_BT_B, NUM_S), lambda i, j: (i, 0)),
        out_shape=jax.ShapeDtypeStruct((B, NUM_S), jnp.int32),
        scratch_shapes=[pltpu.VMEM((_NCB, _CB, _BT_B), jnp.float32)],
    )(cls, dml)


_NW = 32
_BPW = B // _NW
_CH = 8
_NCHUNK = _BPW // _CH
_ROWS = _CH * NUM_S
_LC = OT_DIM // 16


def _gather_sum(OTinput, OToutput, idx3d):
    mesh = plsc.VectorSubcoreMesh(core_axis_name="c", subcore_axis_name="s")

    @functools.partial(
        pl.kernel,
        mesh=mesh,
        out_type=jax.ShapeDtypeStruct((B, OT_DIM), jnp.float32),
        scratch_types=[
            pltpu.VMEM((_NCHUNK, _ROWS), jnp.int32),
            pltpu.VMEM((_ROWS, OT_DIM), jnp.float32),
            pltpu.VMEM((_BPW, OT_DIM), jnp.float32),
            pltpu.SemaphoreType.DMA,
        ],
    )
    def k(otin_hbm, otout_hbm, idx_hbm, out_hbm, idx_v, rows_v, acc_v, sem):
        wid = lax.axis_index("c") * 16 + lax.axis_index("s")
        pltpu.sync_copy(idx_hbm.at[wid], idx_v)

        def zero_row(i, carry):
            for j in range(_LC):
                acc_v[i, pl.ds(j * 16, 16)] = jnp.zeros((16,), jnp.float32)
            return carry

        lax.fori_loop(0, _BPW, zero_row, 0)

        for tbl in (otin_hbm, otout_hbm):
            def chunk_body(c, carry, tbl=tbl):
                pltpu.async_copy(tbl.at[idx_v.at[c]], rows_v, sem).wait()

                def q_body(b, carry2):
                    r0 = b * NUM_S
                    for j in range(_LC):
                        sl = pl.ds(j * 16, 16)
                        v = rows_v[r0, sl]
                        for r in range(1, NUM_S):
                            v = v + rows_v[r0 + r, sl]
                        a = c * _CH + b
                        acc_v[a, sl] = acc_v[a, sl] + v
                    return carry2

                lax.fori_loop(0, _CH, q_body, 0)
                return carry

            lax.fori_loop(0, _NCHUNK, chunk_body, 0)

        pltpu.sync_copy(acc_v, out_hbm.at[pl.ds(wid * _BPW, _BPW)])

    return k(OTinput, OToutput, idx3d)


_NT_D = 1280


def _proj_body(emb_ref, nbr_ref, w1_ref, w2_ref, out_ref):
    out_ref[...] = (
        jnp.dot(emb_ref[...], w1_ref[...], preferred_element_type=jnp.float32)
        + jnp.dot(nbr_ref[...], w2_ref[...], preferred_element_type=jnp.float32)
    )


def _proj(embed, nbr, W_cip1, W_cip2):
    return pl.pallas_call(
        _proj_body,
        grid=(OUT_DIM // _NT_D,),
        in_specs=[
            pl.BlockSpec((B, EMBED_DIM), lambda j: (0, 0)),
            pl.BlockSpec((B, OT_DIM), lambda j: (0, 0)),
            pl.BlockSpec((EMBED_DIM, _NT_D), lambda j: (0, j)),
            pl.BlockSpec((OT_DIM, _NT_D), lambda j: (0, j)),
        ],
        out_specs=pl.BlockSpec((B, _NT_D), lambda j: (0, j)),
        out_shape=jax.ShapeDtypeStruct((B, OUT_DIM), jnp.float32),
    )(embed, nbr, W_cip1, W_cip2)


def kernel(x, dmlfeature, OTinput, OToutput, W_dml, W_emb, W_cip1, W_cip2):
    xf = x.reshape(B, IN_FLAT)
    cls, embed = _prep(xf, W_dml, W_emb)
    idx = _topk(cls, dmlfeature)
    idx3d = idx.reshape(_NW, _NCHUNK, _ROWS)
    nbr = _gather_sum(OTinput, OToutput, idx3d)
    out = _proj(embed, nbr, W_cip1, W_cip2)
    return out.reshape(B, 65, OUT_DIM // 65)
